# baseline (device time: 60354 ns/iter reference)
import jax
import jax.numpy as jnp
from jax import lax
from jax.experimental import pallas as pl
from jax.experimental.pallas import tpu as pltpu

N_DEV = 16
N_TOK = 1024
D_IN = 256
D_OUT = 512
N_EXP = 64
E_PER = N_EXP // N_DEV
CAP = 12
CHUNK = N_TOK // N_DEV
N_HOP = N_DEV - 1


def kernel(x, router_W, route_idx, expert_W):
    del router_W

    def body(x_ref, idx_ref, w_ref, out_ref,
             partial_ref, send_buf, recv_buf, send_sems, recv_sems):
        my = lax.axis_index("i")
        left = lax.rem(my + N_DEV - 1, N_DEV)
        right = lax.rem(my + 1, N_DEV)

        barrier = pltpu.get_barrier_semaphore()
        for nbr in (left, right):
            pl.semaphore_signal(barrier, inc=1, device_id=(nbr,),
                                device_id_type=pl.DeviceIdType.MESH)
        pl.semaphore_wait(barrier, 2)

        e = idx_ref[...]
        cols = lax.broadcasted_iota(jnp.int32, (N_TOK, N_EXP), 1)
        onehot = (e == cols).astype(jnp.float32)
        r = lax.broadcasted_iota(jnp.int32, (N_TOK, N_TOK), 0)
        c = lax.broadcasted_iota(jnp.int32, (N_TOK, N_TOK), 1)
        tril = (r > c).astype(jnp.float32)
        rank = lax.dot_general(tril, onehot, (((1,), (0,)), ((), ())),
                               preferred_element_type=jnp.float32)
        keep = jnp.sum(onehot * (rank < CAP).astype(jnp.float32),
                       axis=1, keepdims=True)

        xv = x_ref[...]
        acc = jnp.zeros((N_TOK, D_OUT), jnp.float32)
        for k in range(E_PER):
            eg = my * E_PER + k
            m = (e == eg).astype(jnp.float32) * keep
            xm = (xv * m).astype(jnp.bfloat16)
            wk = w_ref[k].astype(jnp.bfloat16)
            acc = acc + lax.dot_general(xm, wk, (((1,), (0,)), ((), ())),
                                        preferred_element_type=jnp.float32)
        partial_ref[...] = acc

        def chunk(ci):
            return partial_ref[pl.ds(ci * CHUNK, CHUNK), :]

        rdmas = []
        for h in range(N_HOP):
            rdma = pltpu.make_async_remote_copy(
                src_ref=send_buf.at[h],
                dst_ref=recv_buf.at[h],
                send_sem=send_sems.at[h],
                recv_sem=recv_sems.at[h],
                device_id=(right,),
                device_id_type=pl.DeviceIdType.MESH,
            )
            if h == 0:
                send_buf[0] = chunk(lax.rem(my + N_DEV - 1, N_DEV))
            else:
                rdmas[h - 1].wait_recv()
                ci = lax.rem(my + 2 * N_DEV - 1 - h, N_DEV)
                send_buf[h] = recv_buf[h - 1] + chunk(ci)
            rdma.start()
            rdmas.append(rdma)

        rdmas[N_HOP - 1].wait_recv()
        out_ref[...] = recv_buf[N_HOP - 1] + chunk(my)
        for rdma in rdmas:
            rdma.wait_send()

    return pl.pallas_call(
        body,
        out_shape=jax.ShapeDtypeStruct((CHUNK, D_OUT), jnp.float32),
        in_specs=[pl.BlockSpec(memory_space=pltpu.VMEM)] * 3,
        out_specs=pl.BlockSpec(memory_space=pltpu.VMEM),
        scratch_shapes=[
            pltpu.VMEM((N_TOK, D_OUT), jnp.float32),
            pltpu.VMEM((N_HOP, CHUNK, D_OUT), jnp.float32),
            pltpu.VMEM((N_HOP, CHUNK, D_OUT), jnp.float32),
            pltpu.SemaphoreType.DMA((N_HOP,)),
            pltpu.SemaphoreType.DMA((N_HOP,)),
        ],
        compiler_params=pltpu.CompilerParams(collective_id=0),
    )(x, route_idx, expert_W)


# device time: 34633 ns/iter; 1.7427x vs baseline; 1.7427x over previous
import jax
import jax.numpy as jnp
from jax import lax
from jax.experimental import pallas as pl
from jax.experimental.pallas import tpu as pltpu

N_DEV = 16
N_TOK = 1024
D_IN = 256
D_OUT = 512
N_EXP = 64
E_PER = N_EXP // N_DEV
CAP = 12
CHUNK = N_TOK // N_DEV
HALVES = (8, 4, 2, 1)


def kernel(x, router_W, route_idx, expert_W):
    del router_W

    def body(x_ref, idx_ref, w_ref, out_ref, partial_ref,
             sb0, sb1, sb2, sb3, rb0, rb1, rb2, rb3, send_sems, recv_sems):
        my = lax.axis_index("i")
        partners = [jnp.bitwise_xor(my, h) for h in HALVES]

        barrier = pltpu.get_barrier_semaphore()
        for pt in partners:
            pl.semaphore_signal(barrier, inc=1, device_id=(pt,),
                                device_id_type=pl.DeviceIdType.MESH)
        pl.semaphore_wait(barrier, len(partners))

        e = idx_ref[...]
        cols = lax.broadcasted_iota(jnp.int32, (N_TOK, N_EXP), 1)
        onehot = (e == cols)
        oh_bf = onehot.astype(jnp.bfloat16)
        r = lax.broadcasted_iota(jnp.int32, (N_TOK, N_TOK), 0)
        c = lax.broadcasted_iota(jnp.int32, (N_TOK, N_TOK), 1)
        tril = (r > c).astype(jnp.bfloat16)
        rank = lax.dot_general(tril, oh_bf, (((1,), (0,)), ((), ())),
                               preferred_element_type=jnp.float32)
        keep = jnp.sum(onehot * (rank < CAP), axis=1,
                       keepdims=True).astype(jnp.float32)

        xv = x_ref[...]
        acc = jnp.zeros((N_TOK, D_OUT), jnp.float32)
        for k in range(E_PER):
            eg = my * E_PER + k
            m = (e == eg).astype(jnp.float32) * keep
            xm = (xv * m).astype(jnp.bfloat16)
            wk = w_ref[k].astype(jnp.bfloat16)
            acc = acc + lax.dot_general(xm, wk, (((1,), (0,)), ((), ())),
                                        preferred_element_type=jnp.float32)
        partial_ref[...] = acc

        sbufs = (sb0, sb1, sb2, sb3)
        rbufs = (rb0, rb1, rb2, rb3)
        rdmas = []
        for k, h in enumerate(HALVES):
            partner = partners[k]
            keep_base = (my // h) * h
            send_base = (partner // h) * h
            rows = h * CHUNK
            sbufs[k][...] = partial_ref[pl.ds(send_base * CHUNK, rows),
                                        :].astype(jnp.bfloat16)
            rdma = pltpu.make_async_remote_copy(
                src_ref=sbufs[k],
                dst_ref=rbufs[k],
                send_sem=send_sems.at[k],
                recv_sem=recv_sems.at[k],
                device_id=(partner,),
                device_id_type=pl.DeviceIdType.MESH,
            )
            rdma.start()
            rdma.wait_recv()
            sl = pl.ds(keep_base * CHUNK, rows)
            partial_ref[sl, :] = (partial_ref[sl, :]
                                  + rbufs[k][...].astype(jnp.float32))
            rdmas.append(rdma)

        out_ref[...] = partial_ref[pl.ds(my * CHUNK, CHUNK), :]
        for rdma in rdmas:
            rdma.wait_send()

    return pl.pallas_call(
        body,
        out_shape=jax.ShapeDtypeStruct((CHUNK, D_OUT), jnp.float32),
        in_specs=[pl.BlockSpec(memory_space=pltpu.VMEM)] * 3,
        out_specs=pl.BlockSpec(memory_space=pltpu.VMEM),
        scratch_shapes=[
            pltpu.VMEM((N_TOK, D_OUT), jnp.float32),
            pltpu.VMEM((8 * CHUNK, D_OUT), jnp.bfloat16),
            pltpu.VMEM((4 * CHUNK, D_OUT), jnp.bfloat16),
            pltpu.VMEM((2 * CHUNK, D_OUT), jnp.bfloat16),
            pltpu.VMEM((1 * CHUNK, D_OUT), jnp.bfloat16),
            pltpu.VMEM((8 * CHUNK, D_OUT), jnp.bfloat16),
            pltpu.VMEM((4 * CHUNK, D_OUT), jnp.bfloat16),
            pltpu.VMEM((2 * CHUNK, D_OUT), jnp.bfloat16),
            pltpu.VMEM((1 * CHUNK, D_OUT), jnp.bfloat16),
            pltpu.SemaphoreType.DMA((len(HALVES),)),
            pltpu.SemaphoreType.DMA((len(HALVES),)),
        ],
        compiler_params=pltpu.CompilerParams(collective_id=0),
    )(x, route_idx, expert_W)


# device time: 27113 ns/iter; 2.2260x vs baseline; 1.2774x over previous
import jax
import jax.numpy as jnp
from jax import lax
from jax.experimental import pallas as pl
from jax.experimental.pallas import tpu as pltpu

N_DEV = 16
N_TOK = 1024
D_IN = 256
D_OUT = 512
N_EXP = 64
E_PER = N_EXP // N_DEV
CAP = 12
SLOTS = E_PER * CAP
CHUNK = N_TOK // N_DEV
BITS = (8, 4, 2, 1)


def _bit_reverse4(d):
    b0 = lax.rem(d, 2)
    b1 = lax.rem(d // 2, 2)
    b2 = lax.rem(d // 4, 2)
    b3 = lax.rem(d // 8, 2)
    return b0 * 8 + b1 * 4 + b2 * 2 + b3


def kernel(x, router_W, route_idx, expert_W):
    del router_W

    def body(x_ref, idx_ref, w_ref, out_ref, gbuf, rloc_ref,
             send_sems, recv_sems):
        my = lax.axis_index("i")
        rpos = _bit_reverse4(my)
        partners = [jnp.bitwise_xor(my, b) for b in BITS]

        barrier = pltpu.get_barrier_semaphore()
        for pt in partners:
            pl.semaphore_signal(barrier, inc=1, device_id=(pt,),
                                device_id_type=pl.DeviceIdType.MESH)
        pl.semaphore_wait(barrier, len(partners))

        e = idx_ref[...]
        cols = lax.broadcasted_iota(jnp.int32, (N_TOK, N_EXP), 1)
        oh_bf = (e == cols).astype(jnp.bfloat16)
        r = lax.broadcasted_iota(jnp.int32, (N_TOK, N_TOK), 0)
        c = lax.broadcasted_iota(jnp.int32, (N_TOK, N_TOK), 1)
        tril = (r > c).astype(jnp.bfloat16)
        rank = lax.dot_general(tril, oh_bf, (((1,), (0,)), ((), ())),
                               preferred_element_type=jnp.float32)
        rloc_ref[...] = jnp.sum(oh_bf.astype(jnp.float32) * rank, axis=1,
                                keepdims=True)

        eye64 = (lax.broadcasted_iota(jnp.int32, (N_EXP, N_EXP), 0)
                 == lax.broadcasted_iota(jnp.int32, (N_EXP, N_EXP), 1)
                 ).astype(jnp.bfloat16)
        oh_t = lax.dot_general(eye64, oh_bf, (((1,), (1,)), ((), ())),
                               preferred_element_type=jnp.float32)
        rank_t = lax.dot_general(eye64, rank.astype(jnp.bfloat16),
                                 (((1,), (1,)), ((), ())),
                                 preferred_element_type=jnp.float32)
        srow = lax.broadcasted_iota(jnp.int32, (SLOTS, N_EXP), 0)
        ecol = lax.broadcasted_iota(jnp.int32, (SLOTS, N_EXP), 1)
        r48 = (ecol == my * E_PER + srow // CAP).astype(jnp.bfloat16)
        match_e = lax.dot_general(r48, oh_t.astype(jnp.bfloat16),
                                  (((1,), (0,)), ((), ())),
                                  preferred_element_type=jnp.float32)
        rank48 = lax.dot_general(r48, rank_t.astype(jnp.bfloat16),
                                 (((1,), (0,)), ((), ())),
                                 preferred_element_type=jnp.float32)
        slot_of_s = lax.rem(
            lax.broadcasted_iota(jnp.int32, (SLOTS, N_TOK), 0), CAP)
        g_sel = (match_e * (rank48 == slot_of_s.astype(jnp.float32))
                 ).astype(jnp.bfloat16)
        xg = lax.dot_general(g_sel, x_ref[...].astype(jnp.bfloat16),
                             (((1,), (0,)), ((), ())),
                             preferred_element_type=jnp.float32)
        xg_bf = xg.astype(jnp.bfloat16)
        krow = lax.broadcasted_iota(jnp.int32, (SLOTS, 1), 0) // CAP
        y = jnp.zeros((SLOTS, D_OUT), jnp.float32)
        for k in range(E_PER):
            mk = (krow == k).astype(jnp.bfloat16)
            y = y + lax.dot_general(xg_bf * mk,
                                    w_ref[k].astype(jnp.bfloat16),
                                    (((1,), (0,)), ((), ())),
                                    preferred_element_type=jnp.float32)
        gbuf[pl.ds(rpos * SLOTS, SLOTS), :] = y.astype(jnp.bfloat16)

        rdmas = []
        for j, b in enumerate(BITS):
            grp = 1 << j
            base = (rpos // grp) * grp * SLOTS
            rows = grp * SLOTS
            rdma = pltpu.make_async_remote_copy(
                src_ref=gbuf.at[pl.ds(base, rows)],
                dst_ref=gbuf.at[pl.ds(base, rows)],
                send_sem=send_sems.at[j],
                recv_sem=recv_sems.at[j],
                device_id=(partners[j],),
                device_id_type=pl.DeviceIdType.MESH,
            )
            rdma.start()
            rdma.wait_recv()
            rdmas.append(rdma)

        e_loc = idx_ref[pl.ds(my * CHUNK, CHUNK), :]
        rloc_i = rloc_ref[pl.ds(my * CHUNK, CHUNK), :].astype(jnp.int32)
        kept = rloc_i < CAP
        src_row = (_bit_reverse4(e_loc // E_PER) * SLOTS
                   + lax.rem(e_loc, E_PER) * CAP + rloc_i)
        scol = lax.broadcasted_iota(jnp.int32, (CHUNK, N_DEV * SLOTS), 1)
        sel = ((scol == src_row) & kept).astype(jnp.bfloat16)
        out_ref[...] = lax.dot_general(sel, gbuf[...],
                                       (((1,), (0,)), ((), ())),
                                       preferred_element_type=jnp.float32)

        for rdma in rdmas:
            rdma.wait_send()

    return pl.pallas_call(
        body,
        out_shape=jax.ShapeDtypeStruct((CHUNK, D_OUT), jnp.float32),
        in_specs=[pl.BlockSpec(memory_space=pltpu.VMEM)] * 3,
        out_specs=pl.BlockSpec(memory_space=pltpu.VMEM),
        scratch_shapes=[
            pltpu.VMEM((N_DEV * SLOTS, D_OUT), jnp.bfloat16),
            pltpu.VMEM((N_TOK, 1), jnp.float32),
            pltpu.SemaphoreType.DMA((len(BITS),)),
            pltpu.SemaphoreType.DMA((len(BITS),)),
        ],
        compiler_params=pltpu.CompilerParams(collective_id=0),
    )(x, route_idx, expert_W)


# device time: 20385 ns/iter; 2.9607x vs baseline; 1.3300x over previous
import jax
import jax.numpy as jnp
from jax import lax
from jax.experimental import pallas as pl
from jax.experimental.pallas import tpu as pltpu

N_DEV = 16
N_TOK = 1024
D_IN = 256
D_OUT = 512
N_EXP = 64
E_PER = N_EXP // N_DEV
CAP = 12
SLOTS = E_PER * CAP
CHUNK = N_TOK // N_DEV


def kernel(x, router_W, route_idx, expert_W):
    del router_W

    def body(x_ref, idx_ref, w_ref, out_ref, gbuf, ybuf, rloc_ref,
             send_sems, recv_sems):
        my = lax.axis_index("i")

        barrier = pltpu.get_barrier_semaphore()
        for off in range(1, N_DEV):
            pl.semaphore_signal(barrier, inc=1,
                                device_id=(lax.rem(my + off, N_DEV),),
                                device_id_type=pl.DeviceIdType.MESH)
        pl.semaphore_wait(barrier, N_DEV - 1)

        e = idx_ref[...]
        cols = lax.broadcasted_iota(jnp.int32, (N_TOK, N_EXP), 1)
        oh_bf = (e == cols).astype(jnp.bfloat16)
        r = lax.broadcasted_iota(jnp.int32, (N_TOK, N_TOK), 0)
        c = lax.broadcasted_iota(jnp.int32, (N_TOK, N_TOK), 1)
        tril = (r > c).astype(jnp.bfloat16)
        rank = lax.dot_general(tril, oh_bf, (((1,), (0,)), ((), ())),
                               preferred_element_type=jnp.float32)
        rloc_ref[...] = jnp.sum(oh_bf.astype(jnp.float32) * rank, axis=1,
                                keepdims=True)

        eye64 = (lax.broadcasted_iota(jnp.int32, (N_EXP, N_EXP), 0)
                 == lax.broadcasted_iota(jnp.int32, (N_EXP, N_EXP), 1)
                 ).astype(jnp.bfloat16)
        oh_t = lax.dot_general(eye64, oh_bf, (((1,), (1,)), ((), ())),
                               preferred_element_type=jnp.float32)
        rank_t = lax.dot_general(eye64, rank.astype(jnp.bfloat16),
                                 (((1,), (1,)), ((), ())),
                                 preferred_element_type=jnp.float32)
        srow = lax.broadcasted_iota(jnp.int32, (SLOTS, N_EXP), 0)
        ecol = lax.broadcasted_iota(jnp.int32, (SLOTS, N_EXP), 1)
        r48 = (ecol == my * E_PER + srow // CAP).astype(jnp.bfloat16)
        match_e = lax.dot_general(r48, oh_t.astype(jnp.bfloat16),
                                  (((1,), (0,)), ((), ())),
                                  preferred_element_type=jnp.float32)
        rank48 = lax.dot_general(r48, rank_t.astype(jnp.bfloat16),
                                 (((1,), (0,)), ((), ())),
                                 preferred_element_type=jnp.float32)
        slot_of_s = lax.rem(
            lax.broadcasted_iota(jnp.int32, (SLOTS, N_TOK), 0), CAP)
        g_sel = (match_e * (rank48 == slot_of_s.astype(jnp.float32))
                 ).astype(jnp.bfloat16)
        xg = lax.dot_general(g_sel, x_ref[...].astype(jnp.bfloat16),
                             (((1,), (0,)), ((), ())),
                             preferred_element_type=jnp.float32)
        xg_bf = xg.astype(jnp.bfloat16)
        krow = lax.broadcasted_iota(jnp.int32, (SLOTS, 1), 0) // CAP
        y = jnp.zeros((SLOTS, D_OUT), jnp.float32)
        for k in range(E_PER):
            mk = (krow == k).astype(jnp.bfloat16)
            y = y + lax.dot_general(xg_bf * mk,
                                    w_ref[k].astype(jnp.bfloat16),
                                    (((1,), (0,)), ((), ())),
                                    preferred_element_type=jnp.float32)
        y_bf = y.astype(jnp.bfloat16)
        ybuf[...] = y_bf
        gbuf[pl.ds(my * SLOTS, SLOTS), :] = y_bf

        out_rdmas = []
        for off in range(1, N_DEV):
            rdma = pltpu.make_async_remote_copy(
                src_ref=ybuf,
                dst_ref=gbuf.at[pl.ds(my * SLOTS, SLOTS)],
                send_sem=send_sems.at[off - 1],
                recv_sem=recv_sems.at[off - 1],
                device_id=(lax.rem(my + off, N_DEV),),
                device_id_type=pl.DeviceIdType.MESH,
            )
            rdma.start()
            out_rdmas.append(rdma)

        e_loc = idx_ref[pl.ds(my * CHUNK, CHUNK), :]
        rloc_i = rloc_ref[pl.ds(my * CHUNK, CHUNK), :].astype(jnp.int32)
        kept = rloc_i < CAP
        src_row = (e_loc // E_PER) * SLOTS + lax.rem(e_loc, E_PER) * CAP \
            + rloc_i
        scol = lax.broadcasted_iota(jnp.int32, (CHUNK, N_DEV * SLOTS), 1)
        sel = ((scol == src_row) & kept).astype(jnp.bfloat16)

        for off in range(1, N_DEV):
            src_dev = lax.rem(my + N_DEV - off, N_DEV)
            recv = pltpu.make_async_remote_copy(
                src_ref=ybuf,
                dst_ref=gbuf.at[pl.ds(src_dev * SLOTS, SLOTS)],
                send_sem=send_sems.at[off - 1],
                recv_sem=recv_sems.at[off - 1],
                device_id=(src_dev,),
                device_id_type=pl.DeviceIdType.MESH,
            )
            recv.wait_recv()

        out_ref[...] = lax.dot_general(sel, gbuf[...],
                                       (((1,), (0,)), ((), ())),
                                       preferred_element_type=jnp.float32)

        for rdma in out_rdmas:
            rdma.wait_send()

    return pl.pallas_call(
        body,
        out_shape=jax.ShapeDtypeStruct((CHUNK, D_OUT), jnp.float32),
        in_specs=[pl.BlockSpec(memory_space=pltpu.VMEM)] * 3,
        out_specs=pl.BlockSpec(memory_space=pltpu.VMEM),
        scratch_shapes=[
            pltpu.VMEM((N_DEV * SLOTS, D_OUT), jnp.bfloat16),
            pltpu.VMEM((SLOTS, D_OUT), jnp.bfloat16),
            pltpu.VMEM((N_TOK, 1), jnp.float32),
            pltpu.SemaphoreType.DMA((N_DEV - 1,)),
            pltpu.SemaphoreType.DMA((N_DEV - 1,)),
        ],
        compiler_params=pltpu.CompilerParams(collective_id=0),
    )(x, route_idx, expert_W)


# device time: 19763 ns/iter; 3.0539x vs baseline; 1.0315x over previous
import jax
import jax.numpy as jnp
from jax import lax
from jax.experimental import pallas as pl
from jax.experimental.pallas import tpu as pltpu

N_DEV = 16
N_TOK = 1024
D_IN = 256
D_OUT = 512
N_EXP = 64
E_PER = N_EXP // N_DEV
CAP = 12
SLOTS = E_PER * CAP
CHUNK = N_TOK // N_DEV
TB = 128
NB = N_TOK // TB


def kernel(x, router_W, route_idx, expert_W):
    del router_W

    def body(x_ref, idx_ref, w_ref, out_ref, gbuf, ybuf,
             send_sems, recv_sems):
        my = lax.axis_index("i")

        barrier = pltpu.get_barrier_semaphore()
        for off in range(1, N_DEV):
            pl.semaphore_signal(barrier, inc=1,
                                device_id=(lax.rem(my + off, N_DEV),),
                                device_id_type=pl.DeviceIdType.MESH)
        pl.semaphore_wait(barrier, N_DEV - 1)

        e = idx_ref[...]
        cols = lax.broadcasted_iota(jnp.int32, (N_TOK, N_EXP), 1)
        oh_bf = (e == cols).astype(jnp.bfloat16)
        eye64 = (lax.broadcasted_iota(jnp.int32, (N_EXP, N_EXP), 0)
                 == lax.broadcasted_iota(jnp.int32, (N_EXP, N_EXP), 1)
                 ).astype(jnp.bfloat16)
        oh_t = lax.dot_general(eye64, oh_bf, (((1,), (1,)), ((), ())),
                               preferred_element_type=jnp.float32)
        oh_t_bf = oh_t.astype(jnp.bfloat16)

        k4 = lax.broadcasted_iota(jnp.int32, (E_PER, N_EXP), 0)
        e4 = lax.broadcasted_iota(jnp.int32, (E_PER, N_EXP), 1)
        r4 = (e4 == my * E_PER + k4).astype(jnp.bfloat16)
        oh4 = lax.dot_general(r4, oh_t_bf, (((1,), (0,)), ((), ())),
                              preferred_element_type=jnp.float32)
        oh4_bf = oh4.astype(jnp.bfloat16)
        ju = lax.broadcasted_iota(jnp.int32, (TB, TB), 0)
        tu = lax.broadcasted_iota(jnp.int32, (TB, TB), 1)
        u128 = (ju < tu).astype(jnp.bfloat16)
        tb_row = lax.broadcasted_iota(jnp.int32, (N_TOK, NB), 0) // TB
        bsel = (tb_row == lax.broadcasted_iota(jnp.int32, (N_TOK, NB), 1)
                ).astype(jnp.bfloat16)
        bs4 = lax.dot_general(oh4_bf, bsel, (((1,), (0,)), ((), ())),
                              preferred_element_type=jnp.float32)
        ub = (lax.broadcasted_iota(jnp.int32, (NB, NB), 0)
              < lax.broadcasted_iota(jnp.int32, (NB, NB), 1)
              ).astype(jnp.bfloat16)
        prefix4 = lax.dot_general(bs4.astype(jnp.bfloat16), ub,
                                  (((1,), (0,)), ((), ())),
                                  preferred_element_type=jnp.float32)
        pieces = []
        for b in range(NB):
            intra = lax.dot_general(oh4_bf[:, b * TB:(b + 1) * TB], u128,
                                    (((1,), (0,)), ((), ())),
                                    preferred_element_type=jnp.float32)
            pieces.append(intra + prefix4[:, b:b + 1])
        rank4 = jnp.concatenate(pieces, axis=1)

        srow = lax.broadcasted_iota(jnp.int32, (SLOTS, E_PER), 0) // CAP
        kcol = lax.broadcasted_iota(jnp.int32, (SLOTS, E_PER), 1)
        rrep = (srow == kcol).astype(jnp.bfloat16)
        match_e = lax.dot_general(rrep, oh4_bf, (((1,), (0,)), ((), ())),
                                  preferred_element_type=jnp.float32)
        rank48 = lax.dot_general(rrep, rank4.astype(jnp.bfloat16),
                                 (((1,), (0,)), ((), ())),
                                 preferred_element_type=jnp.float32)
        slot_of_s = lax.rem(
            lax.broadcasted_iota(jnp.int32, (SLOTS, N_TOK), 0), CAP)
        g_sel = (match_e * (rank48 == slot_of_s.astype(jnp.float32))
                 ).astype(jnp.bfloat16)
        xg = lax.dot_general(g_sel, x_ref[...].astype(jnp.bfloat16),
                             (((1,), (0,)), ((), ())),
                             preferred_element_type=jnp.float32)
        xg_bf = xg.astype(jnp.bfloat16)
        krow = lax.broadcasted_iota(jnp.int32, (SLOTS, 1), 0) // CAP
        y = jnp.zeros((SLOTS, D_OUT), jnp.float32)
        for k in range(E_PER):
            mk = (krow == k).astype(jnp.bfloat16)
            y = y + lax.dot_general(xg_bf * mk,
                                    w_ref[k].astype(jnp.bfloat16),
                                    (((1,), (0,)), ((), ())),
                                    preferred_element_type=jnp.float32)
        y_bf = y.astype(jnp.bfloat16)
        ybuf[...] = y_bf
        gbuf[pl.ds(my * SLOTS, SLOTS), :] = y_bf

        out_rdmas = []
        for off in range(1, N_DEV):
            rdma = pltpu.make_async_remote_copy(
                src_ref=ybuf,
                dst_ref=gbuf.at[pl.ds(my * SLOTS, SLOTS)],
                send_sem=send_sems.at[off - 1],
                recv_sem=recv_sems.at[off - 1],
                device_id=(lax.rem(my + off, N_DEV),),
                device_id_type=pl.DeviceIdType.MESH,
            )
            rdma.start()
            out_rdmas.append(rdma)

        irow = lax.broadcasted_iota(jnp.int32, (CHUNK, N_TOK), 0)
        jcol = lax.broadcasted_iota(jnp.int32, (CHUNK, N_TOK), 1)
        tril_rows = (jcol < my * CHUNK + irow).astype(jnp.bfloat16)
        rank_chunk = lax.dot_general(tril_rows, oh_bf,
                                     (((1,), (0,)), ((), ())),
                                     preferred_element_type=jnp.float32)
        e_loc = idx_ref[pl.ds(my * CHUNK, CHUNK), :]
        cols_loc = lax.broadcasted_iota(jnp.int32, (CHUNK, N_EXP), 1)
        oh_loc = (e_loc == cols_loc).astype(jnp.float32)
        rloc_i = jnp.sum(oh_loc * rank_chunk, axis=1,
                         keepdims=True).astype(jnp.int32)
        kept = rloc_i < CAP
        src_row = (e_loc // E_PER) * SLOTS + lax.rem(e_loc, E_PER) * CAP \
            + rloc_i
        scol = lax.broadcasted_iota(jnp.int32, (CHUNK, N_DEV * SLOTS), 1)
        sel = ((scol == src_row) & kept).astype(jnp.bfloat16)

        for off in range(1, N_DEV):
            src_dev = lax.rem(my + N_DEV - off, N_DEV)
            recv = pltpu.make_async_remote_copy(
                src_ref=ybuf,
                dst_ref=gbuf.at[pl.ds(src_dev * SLOTS, SLOTS)],
                send_sem=send_sems.at[off - 1],
                recv_sem=recv_sems.at[off - 1],
                device_id=(src_dev,),
                device_id_type=pl.DeviceIdType.MESH,
            )
            recv.wait_recv()

        out_ref[...] = lax.dot_general(sel, gbuf[...],
                                       (((1,), (0,)), ((), ())),
                                       preferred_element_type=jnp.float32)

        for rdma in out_rdmas:
            rdma.wait_send()

    return pl.pallas_call(
        body,
        out_shape=jax.ShapeDtypeStruct((CHUNK, D_OUT), jnp.float32),
        in_specs=[pl.BlockSpec(memory_space=pltpu.VMEM)] * 3,
        out_specs=pl.BlockSpec(memory_space=pltpu.VMEM),
        scratch_shapes=[
            pltpu.VMEM((N_DEV * SLOTS, D_OUT), jnp.bfloat16),
            pltpu.VMEM((SLOTS, D_OUT), jnp.bfloat16),
            pltpu.SemaphoreType.DMA((N_DEV - 1,)),
            pltpu.SemaphoreType.DMA((N_DEV - 1,)),
        ],
        compiler_params=pltpu.CompilerParams(collective_id=0),
    )(x, route_idx, expert_W)


# device time: 18248 ns/iter; 3.3074x vs baseline; 1.0830x over previous
import jax
import jax.numpy as jnp
from jax import lax
from jax.experimental import pallas as pl
from jax.experimental.pallas import tpu as pltpu

N_DEV = 16
N_TOK = 1024
D_IN = 256
D_OUT = 512
N_EXP = 64
E_PER = N_EXP // N_DEV
CAP = 12
SLOTS = E_PER * CAP
CHUNK = N_TOK // N_DEV
TB = 128
NB = N_TOK // TB


def kernel(x, router_W, route_idx, expert_W):
    del router_W

    def body(x_ref, idx_ref, w_ref, out_ref, gbuf, ybuf,
             send_sems, recv_sems):
        my = lax.axis_index("i")

        barrier = pltpu.get_barrier_semaphore()
        for off in range(1, N_DEV):
            pl.semaphore_signal(barrier, inc=1,
                                device_id=(lax.rem(my + off, N_DEV),),
                                device_id_type=pl.DeviceIdType.MESH)

        e = idx_ref[...]
        cols = lax.broadcasted_iota(jnp.int32, (N_TOK, N_EXP), 1)
        oh_bf = (e == cols).astype(jnp.bfloat16)
        eye64 = (lax.broadcasted_iota(jnp.int32, (N_EXP, N_EXP), 0)
                 == lax.broadcasted_iota(jnp.int32, (N_EXP, N_EXP), 1)
                 ).astype(jnp.bfloat16)
        oh_t = lax.dot_general(eye64, oh_bf, (((1,), (1,)), ((), ())),
                               preferred_element_type=jnp.float32)
        oh_t_bf = oh_t.astype(jnp.bfloat16)

        k4 = lax.broadcasted_iota(jnp.int32, (E_PER, N_EXP), 0)
        e4 = lax.broadcasted_iota(jnp.int32, (E_PER, N_EXP), 1)
        r4 = (e4 == my * E_PER + k4).astype(jnp.bfloat16)
        oh4 = lax.dot_general(r4, oh_t_bf, (((1,), (0,)), ((), ())),
                              preferred_element_type=jnp.float32)
        oh4_bf = oh4.astype(jnp.bfloat16)
        ju = lax.broadcasted_iota(jnp.int32, (TB, TB), 0)
        tu = lax.broadcasted_iota(jnp.int32, (TB, TB), 1)
        u128 = (ju < tu).astype(jnp.bfloat16)
        tb_row = lax.broadcasted_iota(jnp.int32, (N_TOK, NB), 0) // TB
        bsel = (tb_row == lax.broadcasted_iota(jnp.int32, (N_TOK, NB), 1)
                ).astype(jnp.bfloat16)
        bs4 = lax.dot_general(oh4_bf, bsel, (((1,), (0,)), ((), ())),
                              preferred_element_type=jnp.float32)
        ub = (lax.broadcasted_iota(jnp.int32, (NB, NB), 0)
              < lax.broadcasted_iota(jnp.int32, (NB, NB), 1)
              ).astype(jnp.bfloat16)
        prefix4 = lax.dot_general(bs4.astype(jnp.bfloat16), ub,
                                  (((1,), (0,)), ((), ())),
                                  preferred_element_type=jnp.float32)
        pieces = []
        for b in range(NB):
            intra = lax.dot_general(oh4_bf[:, b * TB:(b + 1) * TB], u128,
                                    (((1,), (0,)), ((), ())),
                                    preferred_element_type=jnp.float32)
            pieces.append(intra + prefix4[:, b:b + 1])
        rank4 = jnp.concatenate(pieces, axis=1)

        srow = lax.broadcasted_iota(jnp.int32, (SLOTS, E_PER), 0) // CAP
        kcol = lax.broadcasted_iota(jnp.int32, (SLOTS, E_PER), 1)
        rrep = (srow == kcol).astype(jnp.bfloat16)
        match_e = lax.dot_general(rrep, oh4_bf, (((1,), (0,)), ((), ())),
                                  preferred_element_type=jnp.float32)
        rank48 = lax.dot_general(rrep, rank4.astype(jnp.bfloat16),
                                 (((1,), (0,)), ((), ())),
                                 preferred_element_type=jnp.float32)
        slot_of_s = lax.rem(
            lax.broadcasted_iota(jnp.int32, (SLOTS, N_TOK), 0), CAP)
        g_sel = (match_e * (rank48 == slot_of_s.astype(jnp.float32))
                 ).astype(jnp.bfloat16)
        xg = lax.dot_general(g_sel, x_ref[...].astype(jnp.bfloat16),
                             (((1,), (0,)), ((), ())),
                             preferred_element_type=jnp.float32)
        xg_bf = xg.astype(jnp.bfloat16)
        krow = lax.broadcasted_iota(jnp.int32, (SLOTS, 1), 0) // CAP
        y = jnp.zeros((SLOTS, D_OUT), jnp.float32)
        for k in range(E_PER):
            mk = (krow == k).astype(jnp.bfloat16)
            y = y + lax.dot_general(xg_bf * mk,
                                    w_ref[k].astype(jnp.bfloat16),
                                    (((1,), (0,)), ((), ())),
                                    preferred_element_type=jnp.float32)
        y_bf = y.astype(jnp.bfloat16)
        ybuf[...] = y_bf
        gbuf[pl.ds(my * SLOTS, SLOTS), :] = y_bf

        pl.semaphore_wait(barrier, N_DEV - 1)

        out_rdmas = []
        for off in range(1, N_DEV):
            rdma = pltpu.make_async_remote_copy(
                src_ref=ybuf,
                dst_ref=gbuf.at[pl.ds(my * SLOTS, SLOTS)],
                send_sem=send_sems.at[off - 1],
                recv_sem=recv_sems.at[off - 1],
                device_id=(lax.rem(my + off, N_DEV),),
                device_id_type=pl.DeviceIdType.MESH,
            )
            rdma.start()
            out_rdmas.append(rdma)

        irow = lax.broadcasted_iota(jnp.int32, (CHUNK, N_TOK), 0)
        jcol = lax.broadcasted_iota(jnp.int32, (CHUNK, N_TOK), 1)
        tril_rows = (jcol < my * CHUNK + irow).astype(jnp.bfloat16)
        rank_chunk = lax.dot_general(tril_rows, oh_bf,
                                     (((1,), (0,)), ((), ())),
                                     preferred_element_type=jnp.float32)
        e_loc = idx_ref[pl.ds(my * CHUNK, CHUNK), :]
        cols_loc = lax.broadcasted_iota(jnp.int32, (CHUNK, N_EXP), 1)
        oh_loc = (e_loc == cols_loc).astype(jnp.float32)
        rloc_i = jnp.sum(oh_loc * rank_chunk, axis=1,
                         keepdims=True).astype(jnp.int32)
        kept = rloc_i < CAP
        src_row = (e_loc // E_PER) * SLOTS + lax.rem(e_loc, E_PER) * CAP \
            + rloc_i
        scol = lax.broadcasted_iota(jnp.int32, (CHUNK, N_DEV * SLOTS), 1)
        sel = ((scol == src_row) & kept).astype(jnp.bfloat16)

        for off in range(1, N_DEV):
            src_dev = lax.rem(my + N_DEV - off, N_DEV)
            recv = pltpu.make_async_remote_copy(
                src_ref=ybuf,
                dst_ref=gbuf.at[pl.ds(src_dev * SLOTS, SLOTS)],
                send_sem=send_sems.at[off - 1],
                recv_sem=recv_sems.at[off - 1],
                device_id=(src_dev,),
                device_id_type=pl.DeviceIdType.MESH,
            )
            recv.wait_recv()

        out_ref[...] = lax.dot_general(sel, gbuf[...],
                                       (((1,), (0,)), ((), ())),
                                       preferred_element_type=jnp.float32)

        for rdma in out_rdmas:
            rdma.wait_send()

    return pl.pallas_call(
        body,
        out_shape=jax.ShapeDtypeStruct((CHUNK, D_OUT), jnp.float32),
        in_specs=[pl.BlockSpec(memory_space=pltpu.VMEM)] * 3,
        out_specs=pl.BlockSpec(memory_space=pltpu.VMEM),
        scratch_shapes=[
            pltpu.VMEM((N_DEV * SLOTS, D_OUT), jnp.bfloat16),
            pltpu.VMEM((SLOTS, D_OUT), jnp.bfloat16),
            pltpu.SemaphoreType.DMA((N_DEV - 1,)),
            pltpu.SemaphoreType.DMA((N_DEV - 1,)),
        ],
        compiler_params=pltpu.CompilerParams(collective_id=0),
    )(x, route_idx, expert_W)


# device time: 16870 ns/iter; 3.5776x vs baseline; 1.0817x over previous
import jax
import jax.numpy as jnp
from jax import lax
from jax.experimental import pallas as pl
from jax.experimental.pallas import tpu as pltpu

N_DEV = 16
N_TOK = 1024
D_IN = 256
D_OUT = 512
N_EXP = 64
E_PER = N_EXP // N_DEV
CAP = 12
SLOTS = E_PER * CAP
CHUNK = N_TOK // N_DEV
TB = 128
NB = N_TOK // TB
GRP = 4
GROWS = GRP * SLOTS


def kernel(x, router_W, route_idx, expert_W):
    del router_W

    def body(x_ref, idx_ref, w_ref, out_ref, gB, esbuf, ebuf,
             p1send, p1recv, p2send, p2recv):
        my = lax.axis_index("i")
        gpos = lax.rem(my, GRP)
        gbase = my - gpos
        ggrp = my // GRP

        barrier = pltpu.get_barrier_semaphore()
        peers = [gbase + lax.rem(gpos + d, GRP) for d in range(1, GRP)] \
            + [lax.rem(my + 4 * j, N_DEV) for j in range(1, GRP)]
        for pt in peers:
            pl.semaphore_signal(barrier, inc=1, device_id=(pt,),
                                device_id_type=pl.DeviceIdType.MESH)

        e = idx_ref[...]
        cols = lax.broadcasted_iota(jnp.int32, (N_TOK, N_EXP), 1)
        oh_bf = (e == cols).astype(jnp.bfloat16)
        eye64 = (lax.broadcasted_iota(jnp.int32, (N_EXP, N_EXP), 0)
                 == lax.broadcasted_iota(jnp.int32, (N_EXP, N_EXP), 1)
                 ).astype(jnp.bfloat16)
        oh_t = lax.dot_general(eye64, oh_bf, (((1,), (1,)), ((), ())),
                               preferred_element_type=jnp.float32)
        oh_t_bf = oh_t.astype(jnp.bfloat16)

        k4 = lax.broadcasted_iota(jnp.int32, (E_PER, N_EXP), 0)
        e4 = lax.broadcasted_iota(jnp.int32, (E_PER, N_EXP), 1)
        r4 = (e4 == my * E_PER + k4).astype(jnp.bfloat16)
        oh4 = lax.dot_general(r4, oh_t_bf, (((1,), (0,)), ((), ())),
                              preferred_element_type=jnp.float32)
        oh4_bf = oh4.astype(jnp.bfloat16)
        ju = lax.broadcasted_iota(jnp.int32, (TB, TB), 0)
        tu = lax.broadcasted_iota(jnp.int32, (TB, TB), 1)
        u128 = (ju < tu).astype(jnp.bfloat16)
        tb_row = lax.broadcasted_iota(jnp.int32, (N_TOK, NB), 0) // TB
        bsel = (tb_row == lax.broadcasted_iota(jnp.int32, (N_TOK, NB), 1)
                ).astype(jnp.bfloat16)
        bs4 = lax.dot_general(oh4_bf, bsel, (((1,), (0,)), ((), ())),
                              preferred_element_type=jnp.float32)
        ub = (lax.broadcasted_iota(jnp.int32, (NB, NB), 0)
              < lax.broadcasted_iota(jnp.int32, (NB, NB), 1)
              ).astype(jnp.bfloat16)
        prefix4 = lax.dot_general(bs4.astype(jnp.bfloat16), ub,
                                  (((1,), (0,)), ((), ())),
                                  preferred_element_type=jnp.float32)
        pieces = []
        for b in range(NB):
            intra = lax.dot_general(oh4_bf[:, b * TB:(b + 1) * TB], u128,
                                    (((1,), (0,)), ((), ())),
                                    preferred_element_type=jnp.float32)
            pieces.append(intra + prefix4[:, b:b + 1])
        rank4 = jnp.concatenate(pieces, axis=1)

        srow = lax.broadcasted_iota(jnp.int32, (SLOTS, E_PER), 0) // CAP
        kcol = lax.broadcasted_iota(jnp.int32, (SLOTS, E_PER), 1)
        rrep = (srow == kcol).astype(jnp.bfloat16)
        match_e = lax.dot_general(rrep, oh4_bf, (((1,), (0,)), ((), ())),
                                  preferred_element_type=jnp.float32)
        rank48 = lax.dot_general(rrep, rank4.astype(jnp.bfloat16),
                                 (((1,), (0,)), ((), ())),
                                 preferred_element_type=jnp.float32)
        slot_of_s = lax.rem(
            lax.broadcasted_iota(jnp.int32, (SLOTS, N_TOK), 0), CAP)
        g_sel = (match_e * (rank48 == slot_of_s.astype(jnp.float32))
                 ).astype(jnp.bfloat16)
        xg = lax.dot_general(g_sel, x_ref[...].astype(jnp.bfloat16),
                             (((1,), (0,)), ((), ())),
                             preferred_element_type=jnp.float32)
        xg_bf = xg.astype(jnp.bfloat16)
        krow = lax.broadcasted_iota(jnp.int32, (SLOTS, 1), 0) // CAP
        y = jnp.zeros((SLOTS, D_OUT), jnp.float32)
        for k in range(E_PER):
            mk = (krow == k).astype(jnp.bfloat16)
            y = y + lax.dot_general(xg_bf * mk,
                                    w_ref[k].astype(jnp.bfloat16),
                                    (((1,), (0,)), ((), ())),
                                    preferred_element_type=jnp.float32)
        gB[pl.ds(gpos * SLOTS, SLOTS), :] = y.astype(jnp.bfloat16)

        pl.semaphore_wait(barrier, len(peers))

        p1_rdmas = []
        for d in range(1, GRP):
            rdma = pltpu.make_async_remote_copy(
                src_ref=gB.at[pl.ds(gpos * SLOTS, SLOTS)],
                dst_ref=gB.at[pl.ds(gpos * SLOTS, SLOTS)],
                send_sem=p1send.at[d - 1],
                recv_sem=p1recv.at[d - 1],
                device_id=(gbase + lax.rem(gpos + d, GRP),),
                device_id_type=pl.DeviceIdType.MESH,
            )
            rdma.start()
            p1_rdmas.append(rdma)

        irow = lax.broadcasted_iota(jnp.int32, (CHUNK, N_TOK), 0)
        jcol = lax.broadcasted_iota(jnp.int32, (CHUNK, N_TOK), 1)
        cols_loc = lax.broadcasted_iota(jnp.int32, (CHUNK, N_EXP), 1)
        gcol = lax.broadcasted_iota(jnp.int32, (CHUNK, GROWS), 1)
        sels = []
        for j in range(GRP):
            q = lax.rem(my + 4 * j, N_DEV)
            e_q = idx_ref[pl.ds(q * CHUNK, CHUNK), :]
            tril_q = (jcol < q * CHUNK + irow).astype(jnp.bfloat16)
            rank_cq = lax.dot_general(tril_q, oh_bf,
                                      (((1,), (0,)), ((), ())),
                                      preferred_element_type=jnp.float32)
            oh_q = (e_q == cols_loc).astype(jnp.float32)
            rloc = jnp.sum(oh_q * rank_cq, axis=1, keepdims=True)
            rloc_i = rloc.astype(jnp.int32)
            kept = rloc_i < CAP
            mine = (e_q // (E_PER * GRP)) == ggrp
            grow = (lax.rem(e_q // E_PER, GRP) * SLOTS
                    + lax.rem(e_q, E_PER) * CAP + rloc_i)
            sels.append(((gcol == grow) & kept & mine
                         ).astype(jnp.bfloat16))

        for rdma in p1_rdmas:
            rdma.wait_recv()

        gB_all = gB[...]
        p2_rdmas = []
        for j in range(1, GRP):
            env = lax.dot_general(sels[j], gB_all, (((1,), (0,)), ((), ())),
                                  preferred_element_type=jnp.float32)
            esbuf[j - 1, :, :] = env.astype(jnp.bfloat16)
            rdma = pltpu.make_async_remote_copy(
                src_ref=esbuf.at[j - 1],
                dst_ref=ebuf.at[j - 1],
                send_sem=p2send.at[j - 1],
                recv_sem=p2recv.at[j - 1],
                device_id=(lax.rem(my + 4 * j, N_DEV),),
                device_id_type=pl.DeviceIdType.MESH,
            )
            rdma.start()
            p2_rdmas.append(rdma)

        acc = lax.dot_general(sels[0], gB_all, (((1,), (0,)), ((), ())),
                              preferred_element_type=jnp.float32)
        for j in range(1, GRP):
            p2_rdmas[j - 1].wait_recv()
            acc = acc + ebuf[j - 1].astype(jnp.float32)
        out_ref[...] = acc

        for rdma in p1_rdmas + p2_rdmas:
            rdma.wait_send()

    return pl.pallas_call(
        body,
        out_shape=jax.ShapeDtypeStruct((CHUNK, D_OUT), jnp.float32),
        in_specs=[pl.BlockSpec(memory_space=pltpu.VMEM)] * 3,
        out_specs=pl.BlockSpec(memory_space=pltpu.VMEM),
        scratch_shapes=[
            pltpu.VMEM((GROWS, D_OUT), jnp.bfloat16),
            pltpu.VMEM((GRP - 1, CHUNK, D_OUT), jnp.bfloat16),
            pltpu.VMEM((GRP - 1, CHUNK, D_OUT), jnp.bfloat16),
            pltpu.SemaphoreType.DMA((GRP - 1,)),
            pltpu.SemaphoreType.DMA((GRP - 1,)),
            pltpu.SemaphoreType.DMA((GRP - 1,)),
            pltpu.SemaphoreType.DMA((GRP - 1,)),
        ],
        compiler_params=pltpu.CompilerParams(collective_id=0),
    )(x, route_idx, expert_W)
